# baseline (speedup 1.0000x reference)
import functools

import jax
import jax.numpy as jnp
from jax import lax
from jax.experimental import pallas as pl
from jax.experimental.pallas import tpu as pltpu
from jax.experimental.pallas import tpu_sc as plsc

_T, _H, _F, _E, _K = 2048, 768, 1536, 8, 2
_N = _T * _K
_BM = 256
_NP = _N + _E * _BM
_NB = _NP // _BM

_NC, _NS = 2, 16
_NW = _NC * _NS

_DTOK = _T // _NW


def _dispatch_body(x_hbm, pos0_hbm, pos1_hbm, out_hbm, i0_v, i1_v, rows_v,
                   semr, sem0, sem1):
    wid = lax.axis_index("s") * _NC + lax.axis_index("c")
    base = wid * _DTOK
    cp = pltpu.async_copy(x_hbm.at[pl.ds(base, _DTOK)], rows_v, semr)
    pltpu.sync_copy(pos0_hbm.at[pl.ds(base, _DTOK)], i0_v)
    pltpu.sync_copy(pos1_hbm.at[pl.ds(base, _DTOK)], i1_v)
    cp.wait()
    st0 = pltpu.async_copy(rows_v, out_hbm.at[i0_v], sem0)
    st1 = pltpu.async_copy(rows_v, out_hbm.at[i1_v], sem1)
    st0.wait()
    st1.wait()


_dispatch = functools.partial(
    pl.kernel,
    mesh=plsc.VectorSubcoreMesh(core_axis_name="c", subcore_axis_name="s"),
    out_type=jax.ShapeDtypeStruct((_NP, _H), jnp.float32),
    scratch_types=[
        pltpu.VMEM((_DTOK,), jnp.int32),
        pltpu.VMEM((_DTOK,), jnp.int32),
        pltpu.VMEM((_DTOK, _H), jnp.float32),
        pltpu.SemaphoreType.DMA,
        pltpu.SemaphoreType.DMA,
        pltpu.SemaphoreType.DMA,
    ],
)(_dispatch_body)

_CTOK = _T // _NW
_HV = _H // 16


def _combine_body(y_hbm, pos0_hbm, pos1_hbm, out_hbm, i0_v, i1_v, acc_v,
                  r1_v, sem0, sem1):
    wid = lax.axis_index("s") * _NC + lax.axis_index("c")
    base = wid * _CTOK
    pltpu.sync_copy(pos0_hbm.at[pl.ds(base, _CTOK)], i0_v)
    pltpu.sync_copy(pos1_hbm.at[pl.ds(base, _CTOK)], i1_v)
    cp0 = pltpu.async_copy(y_hbm.at[i0_v], acc_v, sem0)
    cp1 = pltpu.async_copy(y_hbm.at[i1_v], r1_v, sem1)
    cp0.wait()
    cp1.wait()

    def _row(j, carry):
        for i in range(_HV):
            s = pl.ds(i * 16, 16)
            acc_v[j, s] = acc_v[j, s] + r1_v[j, s]
        return carry

    lax.fori_loop(0, _CTOK, _row, 0)
    pltpu.sync_copy(acc_v, out_hbm.at[pl.ds(base, _CTOK)])


_combine = functools.partial(
    pl.kernel,
    mesh=plsc.VectorSubcoreMesh(core_axis_name="c", subcore_axis_name="s"),
    out_type=jax.ShapeDtypeStruct((_T, _H), jnp.float32),
    scratch_types=[
        pltpu.VMEM((_CTOK,), jnp.int32),
        pltpu.VMEM((_CTOK,), jnp.int32),
        pltpu.VMEM((_CTOK, _H), jnp.float32),
        pltpu.VMEM((_CTOK, _H), jnp.float32),
        pltpu.SemaphoreType.DMA,
        pltpu.SemaphoreType.DMA,
    ],
)(_combine_body)


def _gemm_body(be_ref, x_ref, w1_ref, w2_ref, y_ref):
    e = be_ref[pl.program_id(0)]
    h = jnp.dot(x_ref[...], w1_ref[e], preferred_element_type=jnp.float32,
                precision=lax.Precision.DEFAULT)
    h = 0.5 * h * (1.0 + lax.erf(h * 0.7071067811865476))
    y_ref[...] = jnp.dot(h.astype(jnp.bfloat16), w2_ref[0],
                         preferred_element_type=jnp.float32,
                         precision=lax.Precision.DEFAULT)


def _grouped_gemm(be, x_padded, W1, W2):
    grid_spec = pltpu.PrefetchScalarGridSpec(
        num_scalar_prefetch=1,
        grid=(_NB,),
        in_specs=[
            pl.BlockSpec((_BM, _H), lambda i, be: (i, 0)),
            pl.BlockSpec((_E, _H, _F), lambda i, be: (0, 0, 0)),
            pl.BlockSpec((1, _F, _H), lambda i, be: (be[i], 0, 0)),
        ],
        out_specs=pl.BlockSpec((_BM, _H), lambda i, be: (i, 0)),
    )
    return pl.pallas_call(
        _gemm_body,
        grid_spec=grid_spec,
        out_shape=jax.ShapeDtypeStruct((_NP, _H), jnp.float32),
        compiler_params=pltpu.CompilerParams(
            vmem_limit_bytes=60 * 1024 * 1024),
    )(be, x_padded, W1, W2.astype(jnp.bfloat16))


def kernel(inputs_shard, expert_index, W1, W2):
    eid = expert_index.reshape(-1).astype(jnp.int32)

    onehot = (eid[:, None] == jnp.arange(_E, dtype=jnp.int32)[None, :])
    csum = jnp.cumsum(onehot.astype(jnp.int32), axis=0)
    sizes = csum[-1]
    rank = jnp.take_along_axis(csum, eid[:, None], axis=1)[:, 0] - 1
    nblk = (sizes + _BM - 1) // _BM
    bstart = jnp.concatenate(
        [jnp.zeros(1, jnp.int32), jnp.cumsum(nblk)[:-1].astype(jnp.int32)])

    j = jnp.arange(_NB, dtype=jnp.int32)
    be = (jnp.sum(bstart[None, :] <= j[:, None], axis=1) - 1).astype(jnp.int32)

    bsel = jnp.sum(onehot.astype(jnp.int32) * bstart[None, :], axis=1)
    q = bsel * _BM + rank
    pos = q.reshape(_T, _K)
    pos0 = pos[:, 0] + 0
    pos1 = pos[:, 1] + 0

    x_padded = _dispatch(inputs_shard, pos0, pos1)
    y_padded = _grouped_gemm(be, x_padded, W1, W2)
    return _combine(y_padded, pos0, pos1)


# speedup vs baseline: 1.1014x; 1.1014x over previous
import functools

import jax
import jax.numpy as jnp
from jax import lax
from jax.experimental import pallas as pl
from jax.experimental.pallas import tpu as pltpu
from jax.experimental.pallas import tpu_sc as plsc

_T, _H, _F, _E, _K = 2048, 768, 1536, 8, 2
_N = _T * _K
_BM = 256
_NP = _N + _E * _BM
_NB = _NP // _BM

_NC, _NS = 2, 16
_NW = _NC * _NS

_DTOK = _T // _NW


def _dispatch_body(x_hbm, pos0_hbm, pos1_hbm, out_hbm, i0_v, i1_v, rows_v,
                   semr, sem0, sem1):
    wid = lax.axis_index("s") * _NC + lax.axis_index("c")
    base = wid * _DTOK
    cp = pltpu.async_copy(x_hbm.at[pl.ds(base, _DTOK)], rows_v, semr)
    pltpu.sync_copy(pos0_hbm.at[pl.ds(base, _DTOK)], i0_v)
    pltpu.sync_copy(pos1_hbm.at[pl.ds(base, _DTOK)], i1_v)
    cp.wait()
    st0 = pltpu.async_copy(rows_v, out_hbm.at[i0_v], sem0)
    st1 = pltpu.async_copy(rows_v, out_hbm.at[i1_v], sem1)
    st0.wait()
    st1.wait()


_dispatch = functools.partial(
    pl.kernel,
    mesh=plsc.VectorSubcoreMesh(core_axis_name="c", subcore_axis_name="s"),
    out_type=jax.ShapeDtypeStruct((_NP, _H), jnp.float32),
    scratch_types=[
        pltpu.VMEM((_DTOK,), jnp.int32),
        pltpu.VMEM((_DTOK,), jnp.int32),
        pltpu.VMEM((_DTOK, _H), jnp.float32),
        pltpu.SemaphoreType.DMA,
        pltpu.SemaphoreType.DMA,
        pltpu.SemaphoreType.DMA,
    ],
)(_dispatch_body)

_CTOK = _T // _NW
_HV = _H // 16


def _combine_body(y_hbm, pos0_hbm, pos1_hbm, out_hbm, i0_v, i1_v, acc_v,
                  r1_v, sem0, sem1):
    wid = lax.axis_index("s") * _NC + lax.axis_index("c")
    base = wid * _CTOK
    pltpu.sync_copy(pos0_hbm.at[pl.ds(base, _CTOK)], i0_v)
    pltpu.sync_copy(pos1_hbm.at[pl.ds(base, _CTOK)], i1_v)
    cp0 = pltpu.async_copy(y_hbm.at[i0_v], acc_v, sem0)
    cp1 = pltpu.async_copy(y_hbm.at[i1_v], r1_v, sem1)
    cp0.wait()
    cp1.wait()

    def _row(j, carry):
        for i in range(_HV):
            s = pl.ds(i * 16, 16)
            acc_v[j, s] = acc_v[j, s] + r1_v[j, s]
        return carry

    lax.fori_loop(0, _CTOK, _row, 0)
    pltpu.sync_copy(acc_v, out_hbm.at[pl.ds(base, _CTOK)])


_combine = functools.partial(
    pl.kernel,
    mesh=plsc.VectorSubcoreMesh(core_axis_name="c", subcore_axis_name="s"),
    out_type=jax.ShapeDtypeStruct((_T, _H), jnp.float32),
    scratch_types=[
        pltpu.VMEM((_CTOK,), jnp.int32),
        pltpu.VMEM((_CTOK,), jnp.int32),
        pltpu.VMEM((_CTOK, _H), jnp.float32),
        pltpu.VMEM((_CTOK, _H), jnp.float32),
        pltpu.SemaphoreType.DMA,
        pltpu.SemaphoreType.DMA,
    ],
)(_combine_body)


def _gemm_body(be_ref, x_ref, w1_ref, w2_ref, y_ref):
    e = be_ref[pl.program_id(0)]
    h = jnp.dot(x_ref[...], w1_ref[e], preferred_element_type=jnp.float32,
                precision=lax.Precision.DEFAULT)
    h = 0.5 * h * (1.0 + lax.erf(h * 0.7071067811865476))
    y_ref[...] = jnp.dot(h, w2_ref[0], preferred_element_type=jnp.float32,
                         precision=lax.Precision.DEFAULT)


def _grouped_gemm(be, x_padded, W1, W2):
    grid_spec = pltpu.PrefetchScalarGridSpec(
        num_scalar_prefetch=1,
        grid=(_NB,),
        in_specs=[
            pl.BlockSpec((_BM, _H), lambda i, be: (i, 0)),
            pl.BlockSpec((_E, _H, _F), lambda i, be: (0, 0, 0)),
            pl.BlockSpec((1, _F, _H), lambda i, be: (be[i], 0, 0)),
        ],
        out_specs=pl.BlockSpec((_BM, _H), lambda i, be: (i, 0)),
    )
    return pl.pallas_call(
        _gemm_body,
        grid_spec=grid_spec,
        out_shape=jax.ShapeDtypeStruct((_NP, _H), jnp.float32),
        compiler_params=pltpu.CompilerParams(
            vmem_limit_bytes=60 * 1024 * 1024),
    )(be, x_padded, W1, W2)


def kernel(inputs_shard, expert_index, W1, W2):
    eid = expert_index.reshape(-1).astype(jnp.int32)

    onehot = (eid[:, None] == jnp.arange(_E, dtype=jnp.int32)[None, :])
    csum = jnp.cumsum(onehot.astype(jnp.int32), axis=0)
    sizes = csum[-1]
    rank = jnp.take_along_axis(csum, eid[:, None], axis=1)[:, 0] - 1
    nblk = (sizes + _BM - 1) // _BM
    bstart = jnp.concatenate(
        [jnp.zeros(1, jnp.int32), jnp.cumsum(nblk)[:-1].astype(jnp.int32)])

    j = jnp.arange(_NB, dtype=jnp.int32)
    be = (jnp.sum(bstart[None, :] <= j[:, None], axis=1) - 1).astype(jnp.int32)

    bsel = jnp.sum(onehot.astype(jnp.int32) * bstart[None, :], axis=1)
    q = bsel * _BM + rank
    pos = q.reshape(_T, _K)
    pos0 = pos[:, 0] + 0
    pos1 = pos[:, 1] + 0

    x_padded = _dispatch(inputs_shard, pos0, pos1)
    y_padded = _grouped_gemm(be, x_padded, W1, W2)
    return _combine(y_padded, pos0, pos1)
